# baseline (speedup 1.0000x reference)
import jax
import jax.numpy as jnp
from jax import lax
from jax.experimental import pallas as pl
from jax.experimental.pallas import tpu as pltpu
from jax.experimental.pallas import tpu_sc as plsc

N = 100000
E = 6400000
K = 3
NCORES = 2
NTILES = 16
NS = 6272
NP = NTILES * NS
C = 2000
EHALF = E // NCORES
ETILE = EHALF // NTILES
NCHUNK = ETILE // C
NSLOT = 4
LANES = 16
MAGIC = 0x5CA1AB00


def _body(prior_ref, src_ref, dst_ref, lgt_ref, final_ref, ep_ref,
          dx_ref, flag_ref,
          P_sh, D_sh, srcv, dstv, lgtv, gatv, pv, qv, dv, dvr, flagv, seenv,
          in_sems, sc_sems, ep_sems, g_sems, fl_sem):
    c = lax.axis_index("c")
    s = lax.axis_index("s")
    nbase = s * NS
    ebase = c * EHALF + s * ETILE

    def in_start(i, b, vref):
        sl = pl.ds(ebase + i * C, C)
        pltpu.async_copy(src_ref.at[sl], srcv[b], in_sems[b])
        pltpu.async_copy(dst_ref.at[sl], dstv[b], in_sems[b])
        pltpu.async_copy(vref.at[sl], lgtv[b], in_sems[b])

    def in_wait(b):
        pltpu.make_async_copy(src_ref.at[pl.ds(0, C)], srcv[b], in_sems[b]).wait()
        pltpu.make_async_copy(dst_ref.at[pl.ds(0, C)], dstv[b], in_sems[b]).wait()
        pltpu.make_async_copy(lgt_ref.at[pl.ds(0, C)], lgtv[b], in_sems[b]).wait()

    def sc_wait(b):
        pltpu.make_async_copy(gatv[b], D_sh.at[dstv[b]], sc_sems[b]).wait()

    def g_start(b):
        pltpu.async_copy(P_sh.at[srcv[b]], gatv[b], g_sems[b])

    def g_wait(b):
        pltpu.make_async_copy(P_sh.at[srcv[b]], gatv[b], g_sems[b]).wait()

    def ep_wait(i, b):
        pltpu.make_async_copy(
            lgtv[b], ep_ref.at[pl.ds(ebase + i * C, C)], ep_sems[b]).wait()

    pltpu.sync_copy(prior_ref.at[pl.ds(nbase, NS)], pv)

    @plsc.parallel_loop(0, NS // LANES, unroll=8)
    def _(j):
        sl = pl.ds(j * LANES, LANES)
        qv[sl] = 1.0 - pv[sl]
        dv[sl] = jnp.zeros((LANES,), jnp.float32)

    pltpu.sync_copy(pv, P_sh.at[pl.ds(nbase, NS)])
    pltpu.sync_copy(dv, D_sh.at[pl.ds(nbase, NS)])

    @pl.when(s == 0)
    def _():
        flagv[pl.ds(0, LANES)] = jnp.zeros((LANES,), jnp.int32)
        for u in range(K):
            pltpu.sync_copy(
                flagv, flag_ref.at[pl.ds((c * K + u) * LANES, LANES)])

    plsc.subcore_barrier()

    for t in range(1, K + 1):
        vref = lgt_ref if t == 1 else ep_ref
        in_start(0, 0, vref)
        in_start(1, 1, vref)
        in_wait(0)
        g_start(0)

        @pl.loop(0, NCHUNK, step=NSLOT)
        def _(i0):
            for b in range(NSLOT):
                i = i0 + b
                b1 = (b + 1) % NSLOT
                b2 = (b + 2) % NSLOT

                @pl.when(i >= 2)
                def _():
                    sc_wait(b2)
                if t == 1:
                    @pl.when(i >= 2)
                    def _():
                        ep_wait(i - 2, b2)

                @pl.when(i + 2 < NCHUNK)
                def _():
                    in_start(i + 2, b2, vref)

                g_wait(b)

                @pl.when(i + 1 < NCHUNK)
                def _():
                    in_wait(b1)
                    g_start(b1)

                if t == 1:
                    @plsc.parallel_loop(0, C // LANES, unroll=16)
                    def _(j):
                        sl = pl.ds(j * LANES, LANES)
                        e = 1.0 / (1.0 + jnp.exp(-lgtv[b][sl]))
                        lgtv[b][sl] = e
                        gatv[b][sl] = e * gatv[b][sl]
                else:
                    @plsc.parallel_loop(0, C // LANES, unroll=16)
                    def _(j):
                        sl = pl.ds(j * LANES, LANES)
                        gatv[b][sl] = lgtv[b][sl] * gatv[b][sl]

                pltpu.async_copy(gatv[b], D_sh.at[dstv[b]], sc_sems[b], add=True)
                if t == 1:
                    pltpu.async_copy(
                        lgtv[b], ep_ref.at[pl.ds(ebase + i * C, C)], ep_sems[b])

        for i in (NCHUNK - 2, NCHUNK - 1):
            sc_wait(i % NSLOT)
            if t == 1:
                ep_wait(i, i % NSLOT)

        plsc.subcore_barrier()

        pltpu.sync_copy(D_sh.at[pl.ds(nbase, NS)], dv)
        pltpu.sync_copy(dv, dx_ref.at[pl.ds(c * NP + nbase, NS)])

        @plsc.parallel_loop(0, NS // LANES, unroll=8)
        def _(j):
            sl = pl.ds(j * LANES, LANES)
            dvr[sl] = jnp.zeros((LANES,), jnp.float32)

        pltpu.sync_copy(dvr, D_sh.at[pl.ds(nbase, NS)])
        plsc.subcore_barrier()

        @pl.when(s == 0)
        def _():
            flagv[pl.ds(0, LANES)] = jnp.full((LANES,), MAGIC + t, jnp.int32)
            pltpu.sync_copy(
                flagv, flag_ref.at[pl.ds((c * K + t - 1) * LANES, LANES)])

        target = MAGIC + t
        seenv[pl.ds(0, LANES)] = jnp.zeros((LANES,), jnp.int32)

        @pl.loop(0, 100)
        def _(j):
            @pl.when(jnp.min(seenv[pl.ds(0, LANES)], axis=0) == 0)
            def _():
                for _u in range(4):
                    pltpu.async_copy(
                        flag_ref.at[pl.ds(((1 - c) * K + t - 1) * LANES, LANES)],
                        flagv, fl_sem).wait()
                x = flagv[pl.ds(0, LANES)]
                seenv[pl.ds(0, LANES)] = jnp.where(
                    x == target, jnp.int32(1), jnp.int32(0))

        pltpu.sync_copy(dx_ref.at[pl.ds((1 - c) * NP + nbase, NS)], dvr)

        @plsc.parallel_loop(0, NS // LANES, unroll=8)
        def _(j):
            sl = pl.ds(j * LANES, LANES)
            d = dv[sl] + dvr[sl]
            d = jnp.minimum(jnp.maximum(d, 0.0), 10.0)
            q = qv[sl]
            pnew = q * (1.0 - jnp.exp(-d))
            pv[sl] = pnew
            qv[sl] = q * (1.0 - pnew)

        if t < K:
            pltpu.sync_copy(pv, P_sh.at[pl.ds(nbase, NS)])
            plsc.subcore_barrier()

    @plsc.parallel_loop(0, NS // LANES, unroll=8)
    def _(j):
        sl = pl.ds(j * LANES, LANES)
        pv[sl] = 1.0 - qv[sl]

    @pl.when(c == 0)
    def _():
        pltpu.sync_copy(pv, final_ref.at[pl.ds(nbase, NS)])


@jax.jit
def kernel(prior_probs, edge_index, raw_edge_logits):
    prior_pad = jnp.pad(prior_probs, (0, NP - N))
    src = edge_index[0]
    dst = edge_index[1]
    mesh = plsc.VectorSubcoreMesh(core_axis_name="c", subcore_axis_name="s")

    def body(prior_r, src_r, dst_r, lgt_r, final_r, ep_r, dx_r, flag_r,
             P_sh, D_sh, *rest):
        srcv = rest[0:NSLOT]
        dstv = rest[NSLOT:2 * NSLOT]
        lgtv = rest[2 * NSLOT:3 * NSLOT]
        gatv = rest[3 * NSLOT:4 * NSLOT]
        k = 4 * NSLOT
        pv, qv, dv, dvr, flagv, seenv = rest[k:k + 6]
        in_sems = rest[k + 6:k + 6 + NSLOT]
        sc_sems = rest[k + 6 + NSLOT:k + 6 + 2 * NSLOT]
        ep_sems = rest[k + 6 + 2 * NSLOT:k + 6 + 3 * NSLOT]
        g_sems = rest[k + 6 + 3 * NSLOT:k + 6 + 4 * NSLOT]
        fl_sem = rest[k + 6 + 4 * NSLOT]
        _body(prior_r, src_r, dst_r, lgt_r, final_r, ep_r, dx_r, flag_r,
              P_sh, D_sh, srcv, dstv, lgtv, gatv, pv, qv, dv, dvr, flagv, seenv,
              in_sems, sc_sems, ep_sems, g_sems, fl_sem)

    final_pad, edge_probs, _, _ = pl.kernel(
        body,
        out_type=[jax.ShapeDtypeStruct((NP,), jnp.float32),
                  jax.ShapeDtypeStruct((E,), jnp.float32),
                  jax.ShapeDtypeStruct((NCORES * NP,), jnp.float32),
                  jax.ShapeDtypeStruct((NCORES * K * LANES,), jnp.int32)],
        mesh=mesh,
        compiler_params=pltpu.CompilerParams(needs_layout_passes=False),
        scratch_types=[
            pltpu.VMEM_SHARED((NP,), jnp.float32),
            pltpu.VMEM_SHARED((NP,), jnp.float32),
            *[pltpu.VMEM((C,), jnp.int32) for _ in range(NSLOT)],
            *[pltpu.VMEM((C,), jnp.int32) for _ in range(NSLOT)],
            *[pltpu.VMEM((C,), jnp.float32) for _ in range(NSLOT)],
            *[pltpu.VMEM((C,), jnp.float32) for _ in range(NSLOT)],
            pltpu.VMEM((NS,), jnp.float32),
            pltpu.VMEM((NS,), jnp.float32),
            pltpu.VMEM((NS,), jnp.float32),
            pltpu.VMEM((NS,), jnp.float32),
            pltpu.VMEM((LANES,), jnp.int32),
            pltpu.VMEM((LANES,), jnp.int32),
            *[pltpu.SemaphoreType.DMA for _ in range(4 * NSLOT + 1)],
        ],
    )(prior_pad, src, dst, raw_edge_logits)
    return final_pad[:N], edge_probs


# speedup vs baseline: 1.0008x; 1.0008x over previous
import jax
import jax.numpy as jnp
from jax import lax
from jax.experimental import pallas as pl
from jax.experimental.pallas import tpu as pltpu
from jax.experimental.pallas import tpu_sc as plsc

N = 100000
E = 6400000
K = 3
NCORES = 2
NTILES = 16
NS = 6272
NP = NTILES * NS
C = 2000
EHALF = E // NCORES
ETILE = EHALF // NTILES
NCHUNK = ETILE // C
NSLOT = 4
LANES = 16
MAGIC = 0x5CA1AB00


def _body(prior_ref, src_ref, dst_ref, lgt_ref, final_ref, ep_ref,
          dx_ref, flag_ref,
          P_sh, D_sh, srcv, dstv, lgtv, gatv, pv, qv, dv, dvr, flagv, seenv,
          in_sems, sc_sems, ep_sems, g_sems, fl_sem):
    c = lax.axis_index("c")
    s = lax.axis_index("s")
    nbase = s * NS
    ebase = c * EHALF + s * ETILE

    def in_start(i, b, vref):
        sl = pl.ds(ebase + i * C, C)
        pltpu.async_copy(src_ref.at[sl], srcv[b], in_sems[b])
        pltpu.async_copy(dst_ref.at[sl], dstv[b], in_sems[b])
        pltpu.async_copy(vref.at[sl], lgtv[b], in_sems[b])

    def in_wait(b):
        pltpu.make_async_copy(src_ref.at[pl.ds(0, C)], srcv[b], in_sems[b]).wait()
        pltpu.make_async_copy(dst_ref.at[pl.ds(0, C)], dstv[b], in_sems[b]).wait()
        pltpu.make_async_copy(lgt_ref.at[pl.ds(0, C)], lgtv[b], in_sems[b]).wait()

    def sc_wait(b):
        pltpu.make_async_copy(gatv[b], D_sh.at[dstv[b]], sc_sems[b]).wait()

    def g_start(b):
        pltpu.async_copy(P_sh.at[srcv[b]], gatv[b], g_sems[b])

    def g_wait(b):
        pltpu.make_async_copy(P_sh.at[srcv[b]], gatv[b], g_sems[b]).wait()

    def ep_wait(i, b):
        pltpu.make_async_copy(
            lgtv[b], ep_ref.at[pl.ds(ebase + i * C, C)], ep_sems[b]).wait()

    pltpu.sync_copy(prior_ref.at[pl.ds(nbase, NS)], pv)

    @plsc.parallel_loop(0, NS // LANES, unroll=8)
    def _(j):
        sl = pl.ds(j * LANES, LANES)
        qv[sl] = 1.0 - pv[sl]
        dv[sl] = jnp.zeros((LANES,), jnp.float32)

    pltpu.sync_copy(pv, P_sh.at[pl.ds(nbase, NS)])
    pltpu.sync_copy(dv, D_sh.at[pl.ds(nbase, NS)])

    @pl.when(s == 0)
    def _():
        flagv[pl.ds(0, LANES)] = jnp.zeros((LANES,), jnp.int32)
        for u in range(K):
            pltpu.sync_copy(
                flagv, flag_ref.at[pl.ds((c * K + u) * LANES, LANES)])

    plsc.subcore_barrier()

    for t in range(1, K + 1):
        vref = lgt_ref if t == 1 else ep_ref
        in_start(0, 0, vref)
        in_start(1, 1, vref)
        in_wait(0)
        g_start(0)

        @pl.loop(0, NCHUNK, step=NSLOT)
        def _(i0):
            for b in range(NSLOT):
                i = i0 + b
                b1 = (b + 1) % NSLOT
                b2 = (b + 2) % NSLOT

                @pl.when(i >= 2)
                def _():
                    sc_wait(b2)
                if t == 1:
                    @pl.when(i >= 2)
                    def _():
                        ep_wait(i - 2, b2)

                @pl.when(i + 2 < NCHUNK)
                def _():
                    in_start(i + 2, b2, vref)

                g_wait(b)

                @pl.when(i + 1 < NCHUNK)
                def _():
                    in_wait(b1)
                    g_start(b1)

                if t == 1:
                    @plsc.parallel_loop(0, C // LANES, unroll=8)
                    def _(j):
                        sl = pl.ds(j * LANES, LANES)
                        e = 1.0 / (1.0 + jnp.exp(-lgtv[b][sl]))
                        lgtv[b][sl] = e
                        gatv[b][sl] = e * gatv[b][sl]
                else:
                    @plsc.parallel_loop(0, C // LANES, unroll=8)
                    def _(j):
                        sl = pl.ds(j * LANES, LANES)
                        gatv[b][sl] = lgtv[b][sl] * gatv[b][sl]

                pltpu.async_copy(gatv[b], D_sh.at[dstv[b]], sc_sems[b], add=True)
                if t == 1:
                    pltpu.async_copy(
                        lgtv[b], ep_ref.at[pl.ds(ebase + i * C, C)], ep_sems[b])

        for i in (NCHUNK - 2, NCHUNK - 1):
            sc_wait(i % NSLOT)
            if t == 1:
                ep_wait(i, i % NSLOT)

        plsc.subcore_barrier()

        pltpu.sync_copy(D_sh.at[pl.ds(nbase, NS)], dv)
        pltpu.sync_copy(dv, dx_ref.at[pl.ds(c * NP + nbase, NS)])

        @plsc.parallel_loop(0, NS // LANES, unroll=8)
        def _(j):
            sl = pl.ds(j * LANES, LANES)
            dvr[sl] = jnp.zeros((LANES,), jnp.float32)

        pltpu.sync_copy(dvr, D_sh.at[pl.ds(nbase, NS)])
        plsc.subcore_barrier()

        @pl.when(s == 0)
        def _():
            flagv[pl.ds(0, LANES)] = jnp.full((LANES,), MAGIC + t, jnp.int32)
            pltpu.sync_copy(
                flagv, flag_ref.at[pl.ds((c * K + t - 1) * LANES, LANES)])

        target = MAGIC + t
        seenv[pl.ds(0, LANES)] = jnp.zeros((LANES,), jnp.int32)

        @pl.loop(0, 100)
        def _(j):
            @pl.when(jnp.min(seenv[pl.ds(0, LANES)], axis=0) == 0)
            def _():
                for _u in range(4):
                    pltpu.async_copy(
                        flag_ref.at[pl.ds(((1 - c) * K + t - 1) * LANES, LANES)],
                        flagv, fl_sem).wait()
                x = flagv[pl.ds(0, LANES)]
                seenv[pl.ds(0, LANES)] = jnp.where(
                    x == target, jnp.int32(1), jnp.int32(0))

        pltpu.sync_copy(dx_ref.at[pl.ds((1 - c) * NP + nbase, NS)], dvr)

        @plsc.parallel_loop(0, NS // LANES, unroll=8)
        def _(j):
            sl = pl.ds(j * LANES, LANES)
            d = dv[sl] + dvr[sl]
            d = jnp.minimum(jnp.maximum(d, 0.0), 10.0)
            q = qv[sl]
            pnew = q * (1.0 - jnp.exp(-d))
            pv[sl] = pnew
            qv[sl] = q * (1.0 - pnew)

        if t < K:
            pltpu.sync_copy(pv, P_sh.at[pl.ds(nbase, NS)])
            plsc.subcore_barrier()

    @plsc.parallel_loop(0, NS // LANES, unroll=8)
    def _(j):
        sl = pl.ds(j * LANES, LANES)
        pv[sl] = 1.0 - qv[sl]

    @pl.when(c == 0)
    def _():
        pltpu.sync_copy(pv, final_ref.at[pl.ds(nbase, NS)])


@jax.jit
def kernel(prior_probs, edge_index, raw_edge_logits):
    prior_pad = jnp.pad(prior_probs, (0, NP - N))
    src = edge_index[0]
    dst = edge_index[1]
    mesh = plsc.VectorSubcoreMesh(core_axis_name="c", subcore_axis_name="s")

    def body(prior_r, src_r, dst_r, lgt_r, final_r, ep_r, dx_r, flag_r,
             P_sh, D_sh, *rest):
        srcv = rest[0:NSLOT]
        dstv = rest[NSLOT:2 * NSLOT]
        lgtv = rest[2 * NSLOT:3 * NSLOT]
        gatv = rest[3 * NSLOT:4 * NSLOT]
        k = 4 * NSLOT
        pv, qv, dv, dvr, flagv, seenv = rest[k:k + 6]
        in_sems = rest[k + 6:k + 6 + NSLOT]
        sc_sems = rest[k + 6 + NSLOT:k + 6 + 2 * NSLOT]
        ep_sems = rest[k + 6 + 2 * NSLOT:k + 6 + 3 * NSLOT]
        g_sems = rest[k + 6 + 3 * NSLOT:k + 6 + 4 * NSLOT]
        fl_sem = rest[k + 6 + 4 * NSLOT]
        _body(prior_r, src_r, dst_r, lgt_r, final_r, ep_r, dx_r, flag_r,
              P_sh, D_sh, srcv, dstv, lgtv, gatv, pv, qv, dv, dvr, flagv, seenv,
              in_sems, sc_sems, ep_sems, g_sems, fl_sem)

    final_pad, edge_probs, _, _ = pl.kernel(
        body,
        out_type=[jax.ShapeDtypeStruct((NP,), jnp.float32),
                  jax.ShapeDtypeStruct((E,), jnp.float32),
                  jax.ShapeDtypeStruct((NCORES * NP,), jnp.float32),
                  jax.ShapeDtypeStruct((NCORES * K * LANES,), jnp.int32)],
        mesh=mesh,
        compiler_params=pltpu.CompilerParams(needs_layout_passes=False),
        scratch_types=[
            pltpu.VMEM_SHARED((NP,), jnp.float32),
            pltpu.VMEM_SHARED((NP,), jnp.float32),
            *[pltpu.VMEM((C,), jnp.int32) for _ in range(NSLOT)],
            *[pltpu.VMEM((C,), jnp.int32) for _ in range(NSLOT)],
            *[pltpu.VMEM((C,), jnp.float32) for _ in range(NSLOT)],
            *[pltpu.VMEM((C,), jnp.float32) for _ in range(NSLOT)],
            pltpu.VMEM((NS,), jnp.float32),
            pltpu.VMEM((NS,), jnp.float32),
            pltpu.VMEM((NS,), jnp.float32),
            pltpu.VMEM((NS,), jnp.float32),
            pltpu.VMEM((LANES,), jnp.int32),
            pltpu.VMEM((LANES,), jnp.int32),
            *[pltpu.SemaphoreType.DMA for _ in range(4 * NSLOT + 1)],
        ],
    )(prior_pad, src, dst, raw_edge_logits)
    return final_pad[:N], edge_probs
